# baseline (speedup 1.0000x reference)
import functools

import jax
import jax.numpy as jnp
import numpy as np
from jax import lax
from jax.experimental import pallas as pl
from jax.experimental.pallas import tpu as pltpu
from jax.experimental.pallas import tpu_sc as plsc

N = 10000
E = 320000
D = 128
DE = 16
H = 4
F = 32
HF = H * F
PW = HF + 16

NC = 2
NS = 16
K = 80
EPT = E // (NC * NS)
NCHUNK = EPT // K
NP = 10240
RPT = NP // NS


_E1T = np.concatenate(
    [np.kron(np.eye(H), np.ones((1, F))), np.zeros((16 - H, HF))], axis=0
).astype(np.float32).T

_NBLK = 400
_EBLK = E // (N // _NBLK)


def _dotT(x, w):
    return lax.dot_general(x, w, (((1,), (1,)), ((), ())),
                           preferred_element_type=jnp.float32)


def _prep_body(x_ref, eg_ref, wr_ref, wk_ref, a1_ref, a2_ref,
               e1t_ref, pns_ref, r_ref, ep_ref):
    e1t = e1t_ref[...]
    pn = _dotT(x_ref[...], wr_ref[...])
    s16 = jnp.dot(pn * a1_ref[...], e1t,
                  preferred_element_type=jnp.float32)
    pns_ref[:, :HF] = pn
    pns_ref[:, HF:] = s16
    r_ref[...] = jnp.dot(pn * a2_ref[...], e1t,
                         preferred_element_type=jnp.float32)
    ep_ref[...] = jnp.dot(eg_ref[...], wk_ref[...],
                          preferred_element_type=jnp.float32)


def _prepare(nodes, edgesP, wr, wkron, a1r, a2r):
    nb = N // _NBLK
    eb = _EBLK // 8
    return pl.pallas_call(
        _prep_body,
        grid=(nb,),
        in_specs=[
            pl.BlockSpec((_NBLK, D), lambda i: (i, 0)),
            pl.BlockSpec((eb, 128), lambda i: (i, 0)),
            pl.BlockSpec((HF, D), lambda i: (0, 0)),
            pl.BlockSpec((128, 128), lambda i: (0, 0)),
            pl.BlockSpec((1, HF), lambda i: (0, 0)),
            pl.BlockSpec((1, HF), lambda i: (0, 0)),
            pl.BlockSpec((HF, 16), lambda i: (0, 0)),
        ],
        out_specs=[
            pl.BlockSpec((_NBLK, PW), lambda i: (i, 0)),
            pl.BlockSpec((_NBLK, 16), lambda i: (i, 0)),
            pl.BlockSpec((eb, 128), lambda i: (i, 0)),
        ],
        out_shape=[
            jax.ShapeDtypeStruct((N, PW), jnp.float32),
            jax.ShapeDtypeStruct((N, 16), jnp.float32),
            jax.ShapeDtypeStruct((E // 8, 128), jnp.float32),
        ],
    )(nodes, edgesP, wr, wkron, a1r, a2r, jnp.asarray(_E1T))


def _edge_body(pns_hbm, r_hbm, e_hbm, send_hbm, recv_hbm,
               acc_out,
               sidx0, sidx1, ridx0, ridx1, rsc0, rsc1,
               pns0, pns1, rrows0, rrows1, erows0, erows1,
               gsem0, gsem1, isem0, isem1, ssem0, ssem1,
               acc_sh):
    c = lax.axis_index("c")
    sid = lax.axis_index("s")
    zero16 = jnp.zeros((16,), jnp.float32)
    NV = PW // 16
    sidx = (sidx0, sidx1)
    ridx = (ridx0, ridx1)
    rsc = (rsc0, rsc1)
    pns = (pns0, pns1)
    rrows = (rrows0, rrows1)
    erows = (erows0, erows1)
    gsem = (gsem0, gsem1)
    isem = (isem0, isem1)
    ssem = (ssem0, ssem1)

    def z_body(i, _):
        pns0[i // NV, pl.ds((i % NV) * 16, 16)] = zero16
        return 0
    lax.fori_loop(0, K * NV, z_body, 0)

    base_n = sid * RPT
    for k in range(RPT // K):
        pltpu.sync_copy(pns0, acc_sh.at[pl.ds(base_n + k * K, K)])
    plsc.subcore_barrier()

    base_e0 = (c * NS + sid) * EPT

    def idx_copies(it, b):
        off = base_e0 + it * K
        return (
            pltpu.make_async_copy(send_hbm.at[pl.ds(off, K)], sidx[b], isem[b]),
            pltpu.make_async_copy(recv_hbm.at[pl.ds(off, K)], ridx[b], isem[b]),
        )

    def gathers(it, b):
        return (
            pltpu.make_async_copy(pns_hbm.at[sidx[b]], pns[b], gsem[b]),
            pltpu.make_async_copy(r_hbm.at[ridx[b]], rrows[b], gsem[b]),
            pltpu.make_async_copy(e_hbm.at[pl.ds((base_e0 + it * K) // 8, K // 8)],
                                  erows[b], gsem[b]),
        )

    def scatter(b):
        return pltpu.make_async_copy(pns[b], acc_sh.at[rsc[b]], ssem[b])

    for d in idx_copies(0, 0):
        d.start()
    for d in idx_copies(1, 1):
        d.start()
    for d in idx_copies(0, 0):
        d.wait()
    for g in gathers(0, 0):
        g.start()

    def do_chunk(it, b):
        ob = 1 - b
        for g in gathers(it, b):
            g.wait()
        @pl.when(it >= 1)
        def _():
            scatter(ob).wait()
        @pl.when(it + 1 < NCHUNK)
        def _():
            for d in idx_copies(it + 1, ob):
                d.wait()
            for g in gathers(it + 1, ob):
                g.start()
        for v in range(K // 16):
            rsc[b][pl.ds(v * 16, 16)] = ridx[b][pl.ds(v * 16, 16)]

        @pl.when(it + 2 < NCHUNK)
        def _():
            for d in idx_copies(it + 2, b):
                d.start()

        def row_body(rr, _):
            for g in range(8):
                e = rr * 8 + g
                m = (pns[b][e, pl.ds(HF, 16)] + rrows[b][e, :]
                     + erows[b][rr, pl.ds(g * 16, 16)])
                m = jnp.maximum(m, m * 0.01)
                w = jnp.exp(m)
                pns[b][e, pl.ds(HF, 16)] = w
                w0 = w[0]
                w1 = w[1]
                w2 = w[2]
                w3 = w[3]
                for j, wj in enumerate((w0, w0, w1, w1, w2, w2, w3, w3)):
                    pns[b][e, pl.ds(j * 16, 16)] = (
                        pns[b][e, pl.ds(j * 16, 16)] * wj)
            return 0
        lax.fori_loop(0, K // 8, row_body, 0)

        pltpu.async_copy(pns[b], acc_sh.at[rsc[b]], ssem[b], add=True)

    def chunk_body(it, _):
        @pl.when(it % 2 == 0)
        def _():
            do_chunk(it, 0)

        @pl.when(it % 2 == 1)
        def _():
            do_chunk(it, 1)
        return 0
    lax.fori_loop(0, NCHUNK, chunk_body, 0)

    scatter((NCHUNK - 1) % 2).wait()

    plsc.subcore_barrier()
    pltpu.sync_copy(acc_sh.at[pl.ds(base_n, RPT)],
                    acc_out.at[c, pl.ds(base_n, RPT)])


_edge_kernel = functools.partial(
    pl.kernel,
    out_type=jax.ShapeDtypeStruct((NC, NP, PW), jnp.float32),
    mesh=plsc.VectorSubcoreMesh(core_axis_name="c", subcore_axis_name="s"),
    compiler_params=pltpu.CompilerParams(use_tc_tiling_on_sc=False),
    scratch_types=[
        pltpu.VMEM((K,), jnp.int32),
        pltpu.VMEM((K,), jnp.int32),
        pltpu.VMEM((K,), jnp.int32),
        pltpu.VMEM((K,), jnp.int32),
        pltpu.VMEM((K,), jnp.int32),
        pltpu.VMEM((K,), jnp.int32),
        pltpu.VMEM((K, PW), jnp.float32),
        pltpu.VMEM((K, PW), jnp.float32),
        pltpu.VMEM((K, 16), jnp.float32),
        pltpu.VMEM((K, 16), jnp.float32),
        pltpu.VMEM((K // 8, 128), jnp.float32),
        pltpu.VMEM((K // 8, 128), jnp.float32),
        pltpu.SemaphoreType.DMA,
        pltpu.SemaphoreType.DMA,
        pltpu.SemaphoreType.DMA,
        pltpu.SemaphoreType.DMA,
        pltpu.SemaphoreType.DMA,
        pltpu.SemaphoreType.DMA,
        pltpu.VMEM_SHARED((NP, PW), jnp.float32),
    ],
)(_edge_body)


_E1 = np.concatenate(
    [np.kron(np.eye(H), np.ones((1, F))), np.zeros((16 - H, HF))], axis=0
).astype(np.float32)


def _final_body(pns_ref, r_ref, acc_ref, e1_ref, lns_ref, lnb_ref, o_ref):
    pn = pns_ref[:, :HF]
    m = pns_ref[:, HF:] + r_ref[...]
    wself = jnp.exp(jnp.maximum(m, m * 0.01))
    acc = acc_ref[0] + acc_ref[1]
    d16 = acc[:, HF:] + wself
    e1 = e1_ref[...]
    w_exp = jnp.dot(wself, e1, preferred_element_type=jnp.float32)
    d_exp = jnp.dot(d16, e1, preferred_element_type=jnp.float32)
    num = acc[:, :HF] + w_exp * pn
    x = num / d_exp
    x = jnp.where(x > 0, x, jnp.exp(jnp.minimum(x, 0.0)) - 1.0)
    mean = jnp.mean(x, axis=-1, keepdims=True)
    xc = x - mean
    var = jnp.mean(xc * xc, axis=-1, keepdims=True)
    o_ref[...] = xc / jnp.sqrt(var + 1e-6) * lns_ref[...] + lnb_ref[...]


def _finalize(pns, r16, acc2, ln_scale, ln_bias):
    blk = 2000
    return pl.pallas_call(
        _final_body,
        grid=(N // blk,),
        in_specs=[
            pl.BlockSpec((blk, PW), lambda i: (i, 0)),
            pl.BlockSpec((blk, 16), lambda i: (i, 0)),
            pl.BlockSpec((NC, blk, PW), lambda i: (0, i, 0)),
            pl.BlockSpec((16, HF), lambda i: (0, 0)),
            pl.BlockSpec((1, HF), lambda i: (0, 0)),
            pl.BlockSpec((1, HF), lambda i: (0, 0)),
        ],
        out_specs=pl.BlockSpec((blk, HF), lambda i: (i, 0)),
        out_shape=jax.ShapeDtypeStruct((N, HF), jnp.float32),
    )(pns, r16, acc2, jnp.asarray(_E1),
      ln_scale.reshape(1, HF), ln_bias.reshape(1, HF))


def kernel(nodes, edges, receivers, senders, W, W_edge, a, ln_scale, ln_bias):
    wr = W.reshape(HF, D)
    a1r = a[:, :F].reshape(1, HF)
    a2r = a[:, F:2 * F].reshape(1, HF)
    a3 = a[:, 2 * F:]
    ce = jnp.einsum('hfd,hf->dh', W_edge, a3)
    cep = jnp.concatenate([ce, jnp.zeros((DE, 12), jnp.float32)], axis=1)
    wkron = jnp.kron(jnp.eye(8, dtype=jnp.float32), cep)
    edgesP = edges.reshape(E // 8, 128)

    pns, r16, eP = _prepare(nodes, edgesP, wr, wkron, a1r, a2r)

    acc2 = _edge_kernel(pns, r16, eP, senders, receivers)
    return _finalize(pns, r16, acc2, ln_scale, ln_bias)


# speedup vs baseline: 1.4673x; 1.4673x over previous
import functools

import jax
import jax.numpy as jnp
import numpy as np
from jax import lax
from jax.experimental import pallas as pl
from jax.experimental.pallas import tpu as pltpu
from jax.experimental.pallas import tpu_sc as plsc

N = 10000
E = 320000
D = 128
DE = 16
H = 4
F = 32
HF = H * F
PW = HF + 16

NC = 2
NS = 16
K = 80
EPT = E // (NC * NS)
NCHUNK = EPT // K
NP = 10240
RPT = NP // NS


_E1T = np.concatenate(
    [np.kron(np.eye(H), np.ones((1, F))), np.zeros((16 - H, HF))], axis=0
).astype(np.float32).T

_NBLK = 400
_EBLK = E // (N // _NBLK)


def _dotT(x, w):
    return lax.dot_general(x, w, (((1,), (1,)), ((), ())),
                           preferred_element_type=jnp.float32)


def _prep_body(x_ref, eg_ref, wr_ref, wk_ref, a1_ref, a2_ref,
               e1t_ref, pns_ref, r_ref, ep_ref):
    e1t = e1t_ref[...]
    pn = _dotT(x_ref[...], wr_ref[...])
    s16 = jnp.dot(pn * a1_ref[...], e1t,
                  preferred_element_type=jnp.float32)
    pns_ref[:, :HF] = pn
    pns_ref[:, HF:] = s16
    r_ref[...] = jnp.dot(pn * a2_ref[...], e1t,
                         preferred_element_type=jnp.float32)
    ep_ref[...] = jnp.dot(eg_ref[...], wk_ref[...],
                          preferred_element_type=jnp.float32)


def _prepare(nodes, edgesP, wr, wkron, a1r, a2r):
    nb = N // _NBLK
    eb = _EBLK // 8
    return pl.pallas_call(
        _prep_body,
        grid=(nb,),
        in_specs=[
            pl.BlockSpec((_NBLK, D), lambda i: (i, 0)),
            pl.BlockSpec((eb, 128), lambda i: (i, 0)),
            pl.BlockSpec((HF, D), lambda i: (0, 0)),
            pl.BlockSpec((128, 128), lambda i: (0, 0)),
            pl.BlockSpec((1, HF), lambda i: (0, 0)),
            pl.BlockSpec((1, HF), lambda i: (0, 0)),
            pl.BlockSpec((HF, 16), lambda i: (0, 0)),
        ],
        out_specs=[
            pl.BlockSpec((_NBLK, PW), lambda i: (i, 0)),
            pl.BlockSpec((_NBLK, 16), lambda i: (i, 0)),
            pl.BlockSpec((eb, 128), lambda i: (i, 0)),
        ],
        out_shape=[
            jax.ShapeDtypeStruct((N, PW), jnp.float32),
            jax.ShapeDtypeStruct((N, 16), jnp.float32),
            jax.ShapeDtypeStruct((E // 8, 128), jnp.float32),
        ],
    )(nodes, edgesP, wr, wkron, a1r, a2r, jnp.asarray(_E1T))


def _edge_body(pns_hbm, r_hbm, e_hbm, send_hbm, recv_hbm,
               acc_out,
               sidx0, sidx1, ridx0, ridx1, rsc0, rsc1,
               pns0, pns1, rrows0, rrows1, erows0, erows1,
               gsem0, gsem1, isem0, isem1, ssem0, ssem1,
               acc_sh):
    c = lax.axis_index("c")
    sid = lax.axis_index("s")
    zero16 = jnp.zeros((16,), jnp.float32)
    NV = PW // 16
    sidx = (sidx0, sidx1)
    ridx = (ridx0, ridx1)
    rsc = (rsc0, rsc1)
    pns = (pns0, pns1)
    rrows = (rrows0, rrows1)
    erows = (erows0, erows1)
    gsem = (gsem0, gsem1)
    isem = (isem0, isem1)
    ssem = (ssem0, ssem1)

    def z_body(i, _):
        pns0[i // NV, pl.ds((i % NV) * 16, 16)] = zero16
        return 0
    lax.fori_loop(0, K * NV, z_body, 0)

    base_n = sid * RPT
    for k in range(RPT // K):
        pltpu.sync_copy(pns0, acc_sh.at[pl.ds(base_n + k * K, K)])
    plsc.subcore_barrier()

    base_e0 = (c * NS + sid) * EPT

    def idx_copies(it, b):
        off = base_e0 + it * K
        return (
            pltpu.make_async_copy(send_hbm.at[pl.ds(off, K)], sidx[b], isem[b]),
            pltpu.make_async_copy(recv_hbm.at[pl.ds(off, K)], ridx[b], isem[b]),
        )

    def gathers(it, b):
        return (
            pltpu.make_async_copy(pns_hbm.at[sidx[b]], pns[b], gsem[b]),
            pltpu.make_async_copy(r_hbm.at[ridx[b]], rrows[b], gsem[b]),
            pltpu.make_async_copy(e_hbm.at[pl.ds(base_e0 + it * K, K)],
                                  erows[b], gsem[b]),
        )

    def scatter(b):
        return pltpu.make_async_copy(pns[b], acc_sh.at[rsc[b]], ssem[b])

    for d in idx_copies(0, 0):
        d.start()
    for d in idx_copies(1, 1):
        d.start()
    for d in idx_copies(0, 0):
        d.wait()
    for g in gathers(0, 0):
        g.start()

    def do_chunk(it, b):
        ob = 1 - b
        for g in gathers(it, b):
            g.wait()
        @pl.when(it >= 1)
        def _():
            scatter(ob).wait()
        @pl.when(it + 1 < NCHUNK)
        def _():
            for d in idx_copies(it + 1, ob):
                d.wait()
            for g in gathers(it + 1, ob):
                g.start()
        for v in range(K // 16):
            rsc[b][pl.ds(v * 16, 16)] = ridx[b][pl.ds(v * 16, 16)]

        @pl.when(it + 2 < NCHUNK)
        def _():
            for d in idx_copies(it + 2, b):
                d.start()

        def edge_body(e, _):
            m = pns[b][e, pl.ds(HF, 16)] + rrows[b][e, :] + erows[b][e, :]
            m = jnp.maximum(m, m * 0.01)
            w = jnp.exp(m)
            pns[b][e, pl.ds(HF, 16)] = w
            w0 = w[0]
            w1 = w[1]
            w2 = w[2]
            w3 = w[3]
            for j, wj in enumerate((w0, w0, w1, w1, w2, w2, w3, w3)):
                pns[b][e, pl.ds(j * 16, 16)] = pns[b][e, pl.ds(j * 16, 16)] * wj
            return 0
        lax.fori_loop(0, K, edge_body, 0, unroll=2)

        pltpu.async_copy(pns[b], acc_sh.at[rsc[b]], ssem[b], add=True)

    def chunk_body(it, _):
        @pl.when(it % 2 == 0)
        def _():
            do_chunk(it, 0)

        @pl.when(it % 2 == 1)
        def _():
            do_chunk(it, 1)
        return 0
    lax.fori_loop(0, NCHUNK, chunk_body, 0)

    scatter((NCHUNK - 1) % 2).wait()

    plsc.subcore_barrier()
    pltpu.sync_copy(acc_sh.at[pl.ds(base_n, RPT)],
                    acc_out.at[c, pl.ds(base_n, RPT)])


_edge_kernel = functools.partial(
    pl.kernel,
    out_type=jax.ShapeDtypeStruct((NC, NP, PW), jnp.float32),
    mesh=plsc.VectorSubcoreMesh(core_axis_name="c", subcore_axis_name="s"),
    compiler_params=pltpu.CompilerParams(use_tc_tiling_on_sc=False),
    scratch_types=[
        pltpu.VMEM((K,), jnp.int32),
        pltpu.VMEM((K,), jnp.int32),
        pltpu.VMEM((K,), jnp.int32),
        pltpu.VMEM((K,), jnp.int32),
        pltpu.VMEM((K,), jnp.int32),
        pltpu.VMEM((K,), jnp.int32),
        pltpu.VMEM((K, PW), jnp.float32),
        pltpu.VMEM((K, PW), jnp.float32),
        pltpu.VMEM((K, 16), jnp.float32),
        pltpu.VMEM((K, 16), jnp.float32),
        pltpu.VMEM((K, 16), jnp.float32),
        pltpu.VMEM((K, 16), jnp.float32),
        pltpu.SemaphoreType.DMA,
        pltpu.SemaphoreType.DMA,
        pltpu.SemaphoreType.DMA,
        pltpu.SemaphoreType.DMA,
        pltpu.SemaphoreType.DMA,
        pltpu.SemaphoreType.DMA,
        pltpu.VMEM_SHARED((NP, PW), jnp.float32),
    ],
)(_edge_body)


_E1 = np.concatenate(
    [np.kron(np.eye(H), np.ones((1, F))), np.zeros((16 - H, HF))], axis=0
).astype(np.float32)


def _final_body(pns_ref, r_ref, acc_ref, e1_ref, lns_ref, lnb_ref, o_ref):
    pn = pns_ref[:, :HF]
    m = pns_ref[:, HF:] + r_ref[...]
    wself = jnp.exp(jnp.maximum(m, m * 0.01))
    acc = acc_ref[0] + acc_ref[1]
    d16 = acc[:, HF:] + wself
    e1 = e1_ref[...]
    w_exp = jnp.dot(wself, e1, preferred_element_type=jnp.float32)
    d_exp = jnp.dot(d16, e1, preferred_element_type=jnp.float32)
    num = acc[:, :HF] + w_exp * pn
    x = num / d_exp
    x = jnp.where(x > 0, x, jnp.exp(jnp.minimum(x, 0.0)) - 1.0)
    mean = jnp.mean(x, axis=-1, keepdims=True)
    xc = x - mean
    var = jnp.mean(xc * xc, axis=-1, keepdims=True)
    o_ref[...] = xc / jnp.sqrt(var + 1e-6) * lns_ref[...] + lnb_ref[...]


def _finalize(pns, r16, acc2, ln_scale, ln_bias):
    blk = 2000
    return pl.pallas_call(
        _final_body,
        grid=(N // blk,),
        in_specs=[
            pl.BlockSpec((blk, PW), lambda i: (i, 0)),
            pl.BlockSpec((blk, 16), lambda i: (i, 0)),
            pl.BlockSpec((NC, blk, PW), lambda i: (0, i, 0)),
            pl.BlockSpec((16, HF), lambda i: (0, 0)),
            pl.BlockSpec((1, HF), lambda i: (0, 0)),
            pl.BlockSpec((1, HF), lambda i: (0, 0)),
        ],
        out_specs=pl.BlockSpec((blk, HF), lambda i: (i, 0)),
        out_shape=jax.ShapeDtypeStruct((N, HF), jnp.float32),
    )(pns, r16, acc2, jnp.asarray(_E1),
      ln_scale.reshape(1, HF), ln_bias.reshape(1, HF))


def kernel(nodes, edges, receivers, senders, W, W_edge, a, ln_scale, ln_bias):
    wr = W.reshape(HF, D)
    a1r = a[:, :F].reshape(1, HF)
    a2r = a[:, F:2 * F].reshape(1, HF)
    a3 = a[:, 2 * F:]
    ce = jnp.einsum('hfd,hf->dh', W_edge, a3)
    cep = jnp.concatenate([ce, jnp.zeros((DE, 12), jnp.float32)], axis=1)
    wkron = jnp.kron(jnp.eye(8, dtype=jnp.float32), cep)
    edgesP = edges.reshape(E // 8, 128)

    pns, r16, eP = _prepare(nodes, edgesP, wr, wkron, a1r, a2r)

    e16 = eP.reshape(E, DE)
    acc2 = _edge_kernel(pns, r16, e16, senders, receivers)
    return _finalize(pns, r16, acc2, ln_scale, ln_bias)
